# baseline (speedup 1.0000x reference)
import functools

import jax
import jax.numpy as jnp
from jax import lax
from jax.experimental import pallas as pl
from jax.experimental.pallas import tpu as pltpu

SMOOTH = 0.1
BS, C, H, W = 4, 96, 384, 384
CB = 8
HB = 128
GB, GH, GC = BS, H // HB, C // CB


def _dice_body(yh_ref, y_ref, o_ref, acc_ref):
    b = pl.program_id(0)
    h = pl.program_id(1)
    c = pl.program_id(2)
    is_first = jnp.logical_and(jnp.logical_and(b == 0, h == 0), c == 0)
    is_last = jnp.logical_and(
        jnp.logical_and(b == GB - 1, h == GH - 1), c == GC - 1)

    @pl.when(is_first)
    def _():
        acc_ref[0] = 0.0
        acc_ref[1] = 0.0

    x = yh_ref[0]
    p = 1.0 / (1.0 + jnp.exp(-x))
    lbl = y_ref[0]
    cids = c * CB + lax.broadcasted_iota(jnp.int32, (CB, HB, W), 0)
    hit = jnp.where(lbl[None, :, :] == cids, p, 0.0)
    acc_ref[0] += jnp.sum(p)
    acc_ref[1] += jnp.sum(hit)

    @pl.when(is_last)
    def _():
        p_sum = acc_ref[0]
        inter = acc_ref[1]
        y_sum = jnp.float32(BS * H * W)
        o_ref[0] = 1.0 - (2.0 * inter + SMOOTH) / (y_sum + p_sum + SMOOTH)


@jax.jit
def kernel(y_hat, y):
    out = pl.pallas_call(
        _dice_body,
        grid=(GB, GH, GC),
        in_specs=[
            pl.BlockSpec((1, CB, HB, W), lambda b, h, c: (b, c, h, 0)),
            pl.BlockSpec((1, HB, W), lambda b, h, c: (b, h, 0)),
        ],
        out_specs=pl.BlockSpec(
            (1,), lambda b, h, c: (0,), memory_space=pltpu.MemorySpace.SMEM),
        out_shape=jax.ShapeDtypeStruct((1,), jnp.float32),
        scratch_shapes=[pltpu.SMEM((2,), jnp.float32)],
    )(y_hat, y)
    return out[0]


# speedup vs baseline: 1.1260x; 1.1260x over previous
import jax
import jax.numpy as jnp
from jax.experimental import pallas as pl
from jax.experimental.pallas import tpu as pltpu

SMOOTH = 0.1
BS, C, H, W = 4, 96, 384, 384
CB = 8
HB = 128
GB, GH, GC = BS, H // HB, C // CB
NPIX = BS * H * W
NUMEL = BS * C * H * W


def _dice_body(yh_ref, y_ref, o_ref, acc_ref):
    b = pl.program_id(0)
    h = pl.program_id(1)
    c = pl.program_id(2)
    is_first = jnp.logical_and(jnp.logical_and(b == 0, h == 0), c == 0)
    is_last = jnp.logical_and(
        jnp.logical_and(b == GB - 1, h == GH - 1), c == GC - 1)

    @pl.when(is_first)
    def _():
        acc_ref[...] = jnp.zeros_like(acc_ref)

    lbl = y_ref[0]
    a_t = None
    a_i = None
    for ci in range(CB):
        t = jnp.tanh(yh_ref[0, ci] * 0.5)
        ti = jnp.where(lbl == c * CB + ci, t, 0.0)
        a_t = t if a_t is None else a_t + t
        a_i = ti if a_i is None else a_i + ti
    acc_ref[0] += a_t
    acc_ref[1] += a_i

    @pl.when(is_last)
    def _():
        t_sum = jnp.sum(acc_ref[0])
        i_sum = jnp.sum(acc_ref[1])
        p_sum = 0.5 * t_sum + 0.5 * NUMEL
        inter = 0.5 * i_sum + 0.5 * NPIX
        o_ref[0] = 1.0 - (2.0 * inter + SMOOTH) / (NPIX + p_sum + SMOOTH)


@jax.jit
def kernel(y_hat, y):
    out = pl.pallas_call(
        _dice_body,
        grid=(GB, GH, GC),
        in_specs=[
            pl.BlockSpec((1, CB, HB, W), lambda b, h, c: (b, c, h, 0)),
            pl.BlockSpec((1, HB, W), lambda b, h, c: (b, h, 0)),
        ],
        out_specs=pl.BlockSpec(
            (1,), lambda b, h, c: (0,), memory_space=pltpu.MemorySpace.SMEM),
        out_shape=jax.ShapeDtypeStruct((1,), jnp.float32),
        scratch_shapes=[pltpu.VMEM((2, HB, W), jnp.float32)],
    )(y_hat, y)
    return out[0]
